# Initial kernel scaffold; baseline (speedup 1.0000x reference)
import jax
import jax.numpy as jnp
from jax.experimental import pallas as pl


def kernel(x, edge_index_curr, W1, b1, W2, b2):
    raise NotImplementedError("write your pallas kernel here")


# speedup vs baseline: 53.6036x; 53.6036x over previous
import functools

import jax
import jax.numpy as jnp
from jax import lax
from jax.experimental import pallas as pl
from jax.experimental.pallas import tpu as pltpu
from jax.experimental.pallas import tpu_sc as plsc

N = 10000
NPAD = 10240
E = 320000
EPAD = 327680
D_IN = 128
D_HID = 16

NC = 2
NS = 16
NW = NC * NS
CHUNK = 2048
ROWS_PER_W = (EPAD // NW) // 128
CHUNKS = ROWS_PER_W // 16
NODES_PER_S = NPAD // NS

_mesh = plsc.VectorSubcoreMesh(core_axis_name="c", subcore_axis_name="s")


@functools.partial(
    pl.kernel,
    out_type=jax.ShapeDtypeStruct((NC, NPAD, D_HID), jnp.float32),
    mesh=_mesh,
    scratch_types=[
        pltpu.VMEM((CHUNK,), jnp.int32),
        pltpu.VMEM((CHUNK, D_HID), jnp.float32),
        pltpu.VMEM_SHARED((NPAD, D_HID), jnp.float32),
    ],
    compiler_params=pltpu.CompilerParams(use_tc_tiling_on_sc=False),
)
def _deg_kernel(dst_hbm, ones_hbm, zeros_hbm, out_hbm, dst_v, ones_v, acc_s):
    c = lax.axis_index("c")
    s = lax.axis_index("s")
    wid = s * NC + c
    sl = pl.ds(s * NODES_PER_S, NODES_PER_S)
    pltpu.sync_copy(zeros_hbm.at[sl], acc_s.at[sl])
    pltpu.sync_copy(ones_hbm, ones_v)
    plsc.subcore_barrier()
    e0 = wid * (EPAD // NW)
    for k in range(CHUNKS):
        pltpu.sync_copy(dst_hbm.at[pl.ds(e0 + k * CHUNK, CHUNK)], dst_v)
        pltpu.sync_copy(ones_v, acc_s.at[dst_v], add=True)
    plsc.subcore_barrier()
    pltpu.sync_copy(acc_s.at[sl], out_hbm.at[c, sl])


@functools.partial(
    pl.kernel,
    out_type=jax.ShapeDtypeStruct((NC, NPAD, D_HID), jnp.float32),
    mesh=_mesh,
    scratch_types=[
        pltpu.VMEM((CHUNK,), jnp.int32),
        pltpu.VMEM((CHUNK,), jnp.int32),
        pltpu.VMEM((CHUNK, D_HID), jnp.float32),
        pltpu.VMEM_SHARED((NPAD, D_HID), jnp.float32),
        pltpu.SemaphoreType.DMA,
    ],
    compiler_params=pltpu.CompilerParams(use_tc_tiling_on_sc=False),
)
def _agg_kernel(src_hbm, dst_hbm, hs_hbm, zeros_hbm, out_hbm,
                src_v, dst_v, rows_v, acc_s, sem):
    c = lax.axis_index("c")
    s = lax.axis_index("s")
    wid = s * NC + c
    sl = pl.ds(s * NODES_PER_S, NODES_PER_S)
    pltpu.sync_copy(zeros_hbm.at[sl], acc_s.at[sl])
    plsc.subcore_barrier()
    e0 = wid * (EPAD // NW)
    for k in range(CHUNKS):
        pltpu.sync_copy(src_hbm.at[pl.ds(e0 + k * CHUNK, CHUNK)], src_v)
        pltpu.async_copy(hs_hbm.at[src_v], rows_v, sem).wait()
        pltpu.sync_copy(dst_hbm.at[pl.ds(e0 + k * CHUNK, CHUNK)], dst_v)
        pltpu.sync_copy(rows_v, acc_s.at[dst_v], add=True)
    plsc.subcore_barrier()
    pltpu.sync_copy(acc_s.at[sl], out_hbm.at[c, sl])


def _tc1_body(degp_ref, x_ref, w1_ref, hs_ref, dis_ref):
    deg = 1.0 + degp_ref[0] + degp_ref[1]
    dis = lax.rsqrt(deg)
    h = jnp.dot(x_ref[...], w1_ref[...], preferred_element_type=jnp.float32)
    hs_ref[:N, :] = dis[:N, :] * h
    hs_ref[N:, :] = jnp.zeros((NPAD - N, D_HID), jnp.float32)
    dis_ref[...] = dis


def _tc2_body(aggp_ref, hs1_ref, dis_ref, b1_ref, w2_ref, hs2_ref):
    dis = dis_ref[:N, :]
    t = dis * (aggp_ref[0, :N, :] + aggp_ref[1, :N, :] + hs1_ref[:N, :])
    t = jnp.maximum(t + b1_ref[...], 0.0)
    hs2_ref[:N, :] = dis * jnp.dot(t, w2_ref[...],
                                   preferred_element_type=jnp.float32)
    hs2_ref[N:, :] = jnp.zeros((NPAD - N, D_HID), jnp.float32)


def _tc3_body(aggp_ref, hs2_ref, dis_ref, b2_ref, out_ref):
    v = dis_ref[:N, :] * (aggp_ref[0, :N, :] + aggp_ref[1, :N, :]
                          + hs2_ref[:N, :]) + b2_ref[...]
    out_ref[...] = jax.nn.sigmoid(v)


_tc1 = pl.pallas_call(
    _tc1_body,
    out_shape=[jax.ShapeDtypeStruct((NPAD, D_HID), jnp.float32),
               jax.ShapeDtypeStruct((NPAD, D_HID), jnp.float32)],
)

_tc2 = pl.pallas_call(
    _tc2_body,
    out_shape=jax.ShapeDtypeStruct((NPAD, D_HID), jnp.float32),
)

_tc3 = pl.pallas_call(
    _tc3_body,
    out_shape=jax.ShapeDtypeStruct((N, D_HID), jnp.float32),
)


def kernel(x, edge_index_curr, W1, b1, W2, b2):
    src = edge_index_curr[0]
    dst = edge_index_curr[1]
    padv = N + (jnp.arange(EPAD - E, dtype=jnp.int32) % (NPAD - N))
    src1d = jnp.concatenate([src, padv])
    dst1d = jnp.concatenate([dst, padv])

    zeros2d = jnp.zeros((NPAD, D_HID), jnp.float32)
    ones2d = jnp.ones((CHUNK, D_HID), jnp.float32)

    degp = _deg_kernel(dst1d, ones2d, zeros2d)
    hs1, dis = _tc1(degp, x, W1)
    aggp1 = _agg_kernel(src1d, dst1d, hs1, zeros2d)
    hs2 = _tc2(aggp1, hs1, dis, b1.reshape(1, D_HID), W2)
    aggp2 = _agg_kernel(src1d, dst1d, hs2, zeros2d)
    return _tc3(aggp2, hs2, dis, b2.reshape(1, D_HID))


# speedup vs baseline: 61.8029x; 1.1530x over previous
import functools

import jax
import jax.numpy as jnp
from jax import lax
from jax.experimental import pallas as pl
from jax.experimental.pallas import tpu as pltpu
from jax.experimental.pallas import tpu_sc as plsc

N = 10000
NPAD = 10240
E = 320000
EPAD = 327680
D_IN = 128
D_HID = 16

NC = 2
NS = 16
NW = NC * NS
CHUNK = 2048
ROWS_PER_W = (EPAD // NW) // 128
CHUNKS = ROWS_PER_W // 16
NODES_PER_S = NPAD // NS

_mesh = plsc.VectorSubcoreMesh(core_axis_name="c", subcore_axis_name="s")


@functools.partial(
    pl.kernel,
    out_type=jax.ShapeDtypeStruct((NC, NPAD, D_HID), jnp.float32),
    mesh=_mesh,
    scratch_types=[
        [pltpu.VMEM((CHUNK,), jnp.int32) for _ in range(CHUNKS)],
        pltpu.VMEM((CHUNK, D_HID), jnp.float32),
        pltpu.VMEM_SHARED((NPAD, D_HID), jnp.float32),
        pltpu.SemaphoreType.DMA,
    ],
    compiler_params=pltpu.CompilerParams(use_tc_tiling_on_sc=False),
)
def _deg_kernel(dst_hbm, ones_hbm, zeros_hbm, out_hbm, dst_vs, ones_v, acc_s,
                sem):
    c = lax.axis_index("c")
    s = lax.axis_index("s")
    wid = s * NC + c
    sl = pl.ds(s * NODES_PER_S, NODES_PER_S)
    e0 = wid * (EPAD // NW)
    cps = [pltpu.async_copy(dst_hbm.at[pl.ds(e0 + k * CHUNK, CHUNK)],
                            dst_vs[k], sem) for k in range(CHUNKS)]
    one_cp = pltpu.async_copy(ones_hbm, ones_v, sem)
    pltpu.sync_copy(zeros_hbm.at[sl], acc_s.at[sl])
    for cp in cps:
        cp.wait()
    one_cp.wait()
    plsc.subcore_barrier()
    for k in range(CHUNKS):
        pltpu.sync_copy(ones_v, acc_s.at[dst_vs[k]], add=True)
    plsc.subcore_barrier()
    pltpu.sync_copy(acc_s.at[sl], out_hbm.at[c, sl])


@functools.partial(
    pl.kernel,
    out_type=jax.ShapeDtypeStruct((NC, NPAD, D_HID), jnp.float32),
    mesh=_mesh,
    scratch_types=[
        pltpu.VMEM((EPAD // NW,), jnp.int32),
        [pltpu.VMEM((CHUNK,), jnp.int32) for _ in range(CHUNKS)],
        [pltpu.VMEM((CHUNK, D_HID), jnp.float32) for _ in range(2)],
        pltpu.VMEM_SHARED((NPAD, D_HID), jnp.float32),
        pltpu.SemaphoreType.DMA,
        [pltpu.SemaphoreType.DMA for _ in range(2)],
    ],
    compiler_params=pltpu.CompilerParams(use_tc_tiling_on_sc=False),
)
def _agg_kernel(src_hbm, dst_hbm, hs_hbm, zeros_hbm, out_hbm,
                src_all, dst_vs, rows_vs, acc_s, isem, gsems):
    c = lax.axis_index("c")
    s = lax.axis_index("s")
    wid = s * NC + c
    sl = pl.ds(s * NODES_PER_S, NODES_PER_S)
    e0 = wid * (EPAD // NW)
    src_cp = pltpu.async_copy(src_hbm.at[pl.ds(e0, EPAD // NW)], src_all, isem)
    dst_cps = [pltpu.async_copy(dst_hbm.at[pl.ds(e0 + k * CHUNK, CHUNK)],
                                dst_vs[k], isem) for k in range(CHUNKS)]
    pltpu.sync_copy(zeros_hbm.at[sl], acc_s.at[sl])
    src_cp.wait()
    for cp in dst_cps:
        cp.wait()
    plsc.subcore_barrier()
    gcps = [None, None]
    gcps[0] = pltpu.async_copy(hs_hbm.at[src_all.at[pl.ds(0, CHUNK)]],
                               rows_vs[0], gsems[0])
    for k in range(CHUNKS):
        if k + 1 < CHUNKS:
            gcps[(k + 1) % 2] = pltpu.async_copy(
                hs_hbm.at[src_all.at[pl.ds((k + 1) * CHUNK, CHUNK)]],
                rows_vs[(k + 1) % 2], gsems[(k + 1) % 2])
        gcps[k % 2].wait()
        pltpu.sync_copy(rows_vs[k % 2], acc_s.at[dst_vs[k]], add=True)
    plsc.subcore_barrier()
    pltpu.sync_copy(acc_s.at[sl], out_hbm.at[c, sl])


def _tc1_body(degp_ref, x_ref, w1_ref, hs_ref, dis_ref):
    deg = 1.0 + degp_ref[0] + degp_ref[1]
    dis = lax.rsqrt(deg)
    h = jnp.dot(x_ref[...], w1_ref[...], preferred_element_type=jnp.float32)
    hs_ref[:N, :] = dis[:N, :] * h
    hs_ref[N:, :] = jnp.zeros((NPAD - N, D_HID), jnp.float32)
    dis_ref[...] = dis


def _tc2_body(aggp_ref, hs1_ref, dis_ref, b1_ref, w2_ref, hs2_ref):
    dis = dis_ref[:N, :]
    t = dis * (aggp_ref[0, :N, :] + aggp_ref[1, :N, :] + hs1_ref[:N, :])
    t = jnp.maximum(t + b1_ref[...], 0.0)
    hs2_ref[:N, :] = dis * jnp.dot(t, w2_ref[...],
                                   preferred_element_type=jnp.float32)
    hs2_ref[N:, :] = jnp.zeros((NPAD - N, D_HID), jnp.float32)


def _tc3_body(aggp_ref, hs2_ref, dis_ref, b2_ref, out_ref):
    v = dis_ref[:N, :] * (aggp_ref[0, :N, :] + aggp_ref[1, :N, :]
                          + hs2_ref[:N, :]) + b2_ref[...]
    out_ref[...] = jax.nn.sigmoid(v)


_tc1 = pl.pallas_call(
    _tc1_body,
    out_shape=[jax.ShapeDtypeStruct((NPAD, D_HID), jnp.float32),
               jax.ShapeDtypeStruct((NPAD, D_HID), jnp.float32)],
)

_tc2 = pl.pallas_call(
    _tc2_body,
    out_shape=jax.ShapeDtypeStruct((NPAD, D_HID), jnp.float32),
)

_tc3 = pl.pallas_call(
    _tc3_body,
    out_shape=jax.ShapeDtypeStruct((N, D_HID), jnp.float32),
)


def kernel(x, edge_index_curr, W1, b1, W2, b2):
    src = edge_index_curr[0]
    dst = edge_index_curr[1]
    padv = N + (jnp.arange(EPAD - E, dtype=jnp.int32) % (NPAD - N))
    src1d = jnp.concatenate([src, padv])
    dst1d = jnp.concatenate([dst, padv])

    zeros2d = jnp.zeros((NPAD, D_HID), jnp.float32)
    ones2d = jnp.ones((CHUNK, D_HID), jnp.float32)

    degp = _deg_kernel(dst1d, ones2d, zeros2d)
    hs1, dis = _tc1(degp, x, W1)
    aggp1 = _agg_kernel(src1d, dst1d, hs1, zeros2d)
    hs2 = _tc2(aggp1, hs1, dis, b1.reshape(1, D_HID), W2)
    aggp2 = _agg_kernel(src1d, dst1d, hs2, zeros2d)
    return _tc3(aggp2, hs2, dis, b2.reshape(1, D_HID))


# speedup vs baseline: 65.8346x; 1.0652x over previous
import functools

import jax
import jax.numpy as jnp
from jax import lax
from jax.experimental import pallas as pl
from jax.experimental.pallas import tpu as pltpu
from jax.experimental.pallas import tpu_sc as plsc

N = 10000
NPAD = 10240
E = 320000
D_IN = 128
D_HID = 16

NC = 2
NS = 16
NW = NC * NS
EPW = E // NW
CHUNKS = 5
CHUNK = EPW // CHUNKS
NODES_PER_S = NPAD // NS

_mesh = plsc.VectorSubcoreMesh(core_axis_name="c", subcore_axis_name="s")


@functools.partial(
    pl.kernel,
    out_type=jax.ShapeDtypeStruct((NC, NPAD, D_HID), jnp.float32),
    mesh=_mesh,
    scratch_types=[
        [pltpu.VMEM((CHUNK,), jnp.int32) for _ in range(CHUNKS)],
        pltpu.VMEM((CHUNK, D_HID), jnp.float32),
        pltpu.VMEM_SHARED((NPAD, D_HID), jnp.float32),
        pltpu.SemaphoreType.DMA,
    ],
    compiler_params=pltpu.CompilerParams(use_tc_tiling_on_sc=False),
)
def _deg_kernel(ei_hbm, ones_hbm, zeros_hbm, out_hbm, dst_vs, ones_v, acc_s,
                sem):
    c = lax.axis_index("c")
    s = lax.axis_index("s")
    wid = s * NC + c
    sl = pl.ds(s * NODES_PER_S, NODES_PER_S)
    e0 = wid * EPW
    cps = [pltpu.async_copy(ei_hbm.at[1, pl.ds(e0 + k * CHUNK, CHUNK)],
                            dst_vs[k], sem) for k in range(CHUNKS)]
    one_cp = pltpu.async_copy(ones_hbm, ones_v, sem)
    pltpu.sync_copy(zeros_hbm.at[sl], acc_s.at[sl])
    for cp in cps:
        cp.wait()
    one_cp.wait()
    plsc.subcore_barrier()
    for k in range(CHUNKS):
        pltpu.sync_copy(ones_v, acc_s.at[dst_vs[k]], add=True)
    plsc.subcore_barrier()
    pltpu.sync_copy(acc_s.at[sl], out_hbm.at[c, sl])


@functools.partial(
    pl.kernel,
    out_type=jax.ShapeDtypeStruct((NC, NPAD, D_HID), jnp.float32),
    mesh=_mesh,
    scratch_types=[
        pltpu.VMEM((EPW,), jnp.int32),
        [pltpu.VMEM((CHUNK,), jnp.int32) for _ in range(CHUNKS)],
        [pltpu.VMEM((CHUNK, D_HID), jnp.float32) for _ in range(2)],
        pltpu.VMEM_SHARED((NPAD, D_HID), jnp.float32),
        pltpu.SemaphoreType.DMA,
        [pltpu.SemaphoreType.DMA for _ in range(2)],
    ],
    compiler_params=pltpu.CompilerParams(use_tc_tiling_on_sc=False),
)
def _agg_kernel(ei_hbm, hs_hbm, zeros_hbm, out_hbm,
                src_all, dst_vs, rows_vs, acc_s, isem, gsems):
    c = lax.axis_index("c")
    s = lax.axis_index("s")
    wid = s * NC + c
    sl = pl.ds(s * NODES_PER_S, NODES_PER_S)
    e0 = wid * EPW
    src_cp = pltpu.async_copy(ei_hbm.at[0, pl.ds(e0, EPW)], src_all, isem)
    dst_cps = [pltpu.async_copy(ei_hbm.at[1, pl.ds(e0 + k * CHUNK, CHUNK)],
                                dst_vs[k], isem) for k in range(CHUNKS)]
    pltpu.sync_copy(zeros_hbm.at[sl], acc_s.at[sl])
    src_cp.wait()
    for cp in dst_cps:
        cp.wait()
    plsc.subcore_barrier()
    gcps = [None, None]
    gcps[0] = pltpu.async_copy(hs_hbm.at[src_all.at[pl.ds(0, CHUNK)]],
                               rows_vs[0], gsems[0])
    for k in range(CHUNKS):
        if k + 1 < CHUNKS:
            gcps[(k + 1) % 2] = pltpu.async_copy(
                hs_hbm.at[src_all.at[pl.ds((k + 1) * CHUNK, CHUNK)]],
                rows_vs[(k + 1) % 2], gsems[(k + 1) % 2])
        gcps[k % 2].wait()
        pltpu.sync_copy(rows_vs[k % 2], acc_s.at[dst_vs[k]], add=True)
    plsc.subcore_barrier()
    pltpu.sync_copy(acc_s.at[sl], out_hbm.at[c, sl])


def _tcmm_body(x_ref, w1_ref, h_ref):
    h_ref[...] = jnp.dot(x_ref[...], w1_ref[...],
                         preferred_element_type=jnp.float32)


def _tcscale_body(degp_ref, h_ref, hs_ref, dis_ref):
    dis = lax.rsqrt(1.0 + degp_ref[0, :N, :] + degp_ref[1, :N, :])
    hs_ref[...] = dis * h_ref[...]
    dis_ref[...] = dis


def _tc2_body(aggp_ref, hs1_ref, dis_ref, b1_ref, w2_ref, hs2_ref):
    dis = dis_ref[...]
    t = dis * (aggp_ref[0, :N, :] + aggp_ref[1, :N, :] + hs1_ref[...])
    t = jnp.maximum(t + b1_ref[...], 0.0)
    hs2_ref[...] = dis * jnp.dot(t, w2_ref[...],
                                 preferred_element_type=jnp.float32)


def _tc3_body(aggp_ref, hs2_ref, dis_ref, b2_ref, out_ref):
    v = dis_ref[...] * (aggp_ref[0, :N, :] + aggp_ref[1, :N, :]
                        + hs2_ref[...]) + b2_ref[...]
    out_ref[...] = jax.nn.sigmoid(v)


_tcmm = pl.pallas_call(
    _tcmm_body,
    out_shape=jax.ShapeDtypeStruct((N, D_HID), jnp.float32),
)

_tcscale = pl.pallas_call(
    _tcscale_body,
    out_shape=[jax.ShapeDtypeStruct((N, D_HID), jnp.float32),
               jax.ShapeDtypeStruct((N, D_HID), jnp.float32)],
)

_tc2 = pl.pallas_call(
    _tc2_body,
    out_shape=jax.ShapeDtypeStruct((N, D_HID), jnp.float32),
)

_tc3 = pl.pallas_call(
    _tc3_body,
    out_shape=jax.ShapeDtypeStruct((N, D_HID), jnp.float32),
)


def kernel(x, edge_index_curr, W1, b1, W2, b2):
    zeros2d = jnp.zeros((NPAD, D_HID), jnp.float32)
    ones2d = jnp.ones((CHUNK, D_HID), jnp.float32)

    degp = _deg_kernel(edge_index_curr, ones2d, zeros2d)
    h1 = _tcmm(x, W1)
    hs1, dis = _tcscale(degp, h1)
    aggp1 = _agg_kernel(edge_index_curr, hs1, zeros2d)
    hs2 = _tc2(aggp1, hs1, dis, b1.reshape(1, D_HID), W2)
    aggp2 = _agg_kernel(edge_index_curr, hs2, zeros2d)
    return _tc3(aggp2, hs2, dis, b2.reshape(1, D_HID))


# speedup vs baseline: 100.4892x; 1.5264x over previous
import functools

import jax
import jax.numpy as jnp
from jax import lax
from jax.experimental import pallas as pl
from jax.experimental.pallas import tpu as pltpu
from jax.experimental.pallas import tpu_sc as plsc

N = 10000
NPAD = 10240
E = 320000
D_IN = 128
D_HID = 16
PK = 128 // D_HID
NP_P = N // PK
NPAD_P = NPAD // PK

NC = 2
NS = 16
NW = NC * NS
EPW = E // NW
CHUNKS = 5
CHUNK = EPW // CHUNKS
NODES_PER_S = NPAD // NS

_mesh = plsc.VectorSubcoreMesh(core_axis_name="c", subcore_axis_name="s")


@functools.partial(
    pl.kernel,
    out_type=jax.ShapeDtypeStruct((NC, NPAD, D_HID), jnp.float32),
    mesh=_mesh,
    scratch_types=[
        [pltpu.VMEM((CHUNK,), jnp.int32) for _ in range(CHUNKS)],
        pltpu.VMEM((CHUNK, D_HID), jnp.float32),
        pltpu.VMEM_SHARED((NPAD, D_HID), jnp.float32),
        pltpu.SemaphoreType.DMA,
        pltpu.SemaphoreType.DMA,
    ],
    compiler_params=pltpu.CompilerParams(use_tc_tiling_on_sc=False),
)
def _deg_kernel(ei_hbm, ones_hbm, zeros_hbm, out_hbm, dst_vs, ones_v, acc_s,
                isem, ssem):
    c = lax.axis_index("c")
    s = lax.axis_index("s")
    wid = s * NC + c
    sl = pl.ds(s * NODES_PER_S, NODES_PER_S)
    e0 = wid * EPW
    cps = [pltpu.async_copy(ei_hbm.at[1, pl.ds(e0 + k * CHUNK, CHUNK)],
                            dst_vs[k], isem) for k in range(CHUNKS)]
    one_cp = pltpu.async_copy(ones_hbm, ones_v, isem)
    pltpu.sync_copy(zeros_hbm.at[sl], acc_s.at[sl])
    for cp in cps:
        cp.wait()
    one_cp.wait()
    plsc.subcore_barrier()
    scps = [pltpu.async_copy(ones_v, acc_s.at[dst_vs[k]], ssem, add=True)
            for k in range(CHUNKS)]
    for cp in scps:
        cp.wait()
    plsc.subcore_barrier()
    pltpu.sync_copy(acc_s.at[sl], out_hbm.at[c, sl])


@functools.partial(
    pl.kernel,
    out_type=jax.ShapeDtypeStruct((NC, NPAD, D_HID), jnp.float32),
    mesh=_mesh,
    scratch_types=[
        pltpu.VMEM((EPW,), jnp.int32),
        [pltpu.VMEM((CHUNK,), jnp.int32) for _ in range(CHUNKS)],
        [pltpu.VMEM((CHUNK, D_HID), jnp.float32) for _ in range(2)],
        pltpu.VMEM_SHARED((NPAD, D_HID), jnp.float32),
        pltpu.SemaphoreType.DMA,
        [pltpu.SemaphoreType.DMA for _ in range(2)],
    ],
    compiler_params=pltpu.CompilerParams(use_tc_tiling_on_sc=False),
)
def _agg_kernel(ei_hbm, hs_hbm, zeros_hbm, out_hbm,
                src_all, dst_vs, rows_vs, acc_s, isem, gsems):
    c = lax.axis_index("c")
    s = lax.axis_index("s")
    wid = s * NC + c
    sl = pl.ds(s * NODES_PER_S, NODES_PER_S)
    e0 = wid * EPW
    src_cp = pltpu.async_copy(ei_hbm.at[0, pl.ds(e0, EPW)], src_all, isem)
    dst_cps = [pltpu.async_copy(ei_hbm.at[1, pl.ds(e0 + k * CHUNK, CHUNK)],
                                dst_vs[k], isem) for k in range(CHUNKS)]
    pltpu.sync_copy(zeros_hbm.at[sl], acc_s.at[sl])
    src_cp.wait()
    for cp in dst_cps:
        cp.wait()
    plsc.subcore_barrier()
    gcps = [None, None]
    gcps[0] = pltpu.async_copy(hs_hbm.at[src_all.at[pl.ds(0, CHUNK)]],
                               rows_vs[0], gsems[0])
    for k in range(CHUNKS):
        if k + 1 < CHUNKS:
            gcps[(k + 1) % 2] = pltpu.async_copy(
                hs_hbm.at[src_all.at[pl.ds((k + 1) * CHUNK, CHUNK)]],
                rows_vs[(k + 1) % 2], gsems[(k + 1) % 2])
        gcps[k % 2].wait()
        pltpu.sync_copy(rows_vs[k % 2], acc_s.at[dst_vs[k]], add=True)
    plsc.subcore_barrier()
    pltpu.sync_copy(acc_s.at[sl], out_hbm.at[c, sl])


def _tcmm_body(xp_ref, w1b_ref, h_ref):
    h_ref[:NP_P, :] = jnp.dot(xp_ref[...], w1b_ref[...],
                              preferred_element_type=jnp.float32)
    h_ref[NP_P:, :] = jnp.zeros((NPAD_P - NP_P, 128), jnp.float32)


def _tcscale_body(degp_ref, h_ref, hs_ref, dis_ref):
    dis = lax.rsqrt(1.0 + degp_ref[0] + degp_ref[1])
    hs_ref[...] = dis * h_ref[...]
    dis_ref[...] = dis


def _tc2_body(aggp_ref, hs1_ref, dis_ref, b1_ref, w2b_ref, hs2_ref):
    dis = dis_ref[...]
    t = dis * (aggp_ref[0] + aggp_ref[1] + hs1_ref[...])
    t = jnp.maximum(t + b1_ref[...], 0.0)
    u = jnp.dot(t, w2b_ref[...], preferred_element_type=jnp.float32)
    hs2_ref[:NP_P, :] = dis[:NP_P, :] * u[:NP_P, :]
    hs2_ref[NP_P:, :] = jnp.zeros((NPAD_P - NP_P, 128), jnp.float32)


def _tc3_body(aggp_ref, hs2_ref, dis_ref, b2_ref, out_ref):
    v = dis_ref[:NP_P, :] * (aggp_ref[0, :NP_P, :] + aggp_ref[1, :NP_P, :]
                             + hs2_ref[:NP_P, :]) + b2_ref[...]
    out_ref[...] = jax.nn.sigmoid(v)


_tcmm = pl.pallas_call(
    _tcmm_body,
    out_shape=jax.ShapeDtypeStruct((NPAD_P, 128), jnp.float32),
)

_tcscale = pl.pallas_call(
    _tcscale_body,
    out_shape=[jax.ShapeDtypeStruct((NPAD_P, 128), jnp.float32),
               jax.ShapeDtypeStruct((NPAD_P, 128), jnp.float32)],
)

_tc2 = pl.pallas_call(
    _tc2_body,
    out_shape=jax.ShapeDtypeStruct((NPAD_P, 128), jnp.float32),
)

_tc3 = pl.pallas_call(
    _tc3_body,
    out_shape=jax.ShapeDtypeStruct((NP_P, 128), jnp.float32),
)


def kernel(x, edge_index_curr, W1, b1, W2, b2):
    zeros2d = jnp.zeros((NPAD, D_HID), jnp.float32)
    ones2d = jnp.ones((CHUNK, D_HID), jnp.float32)

    w1b = jax.scipy.linalg.block_diag(*([W1] * PK))
    w2b = jax.scipy.linalg.block_diag(*([W2] * PK))
    b1t = jnp.tile(b1, PK).reshape(1, 128)
    b2t = jnp.tile(b2, PK).reshape(1, 128)
    xp = x.reshape(NP_P, PK * D_IN)

    degp = _deg_kernel(edge_index_curr, ones2d, zeros2d)
    h1 = _tcmm(xp, w1b)
    hs1, dis = _tcscale(degp.reshape(NC, NPAD_P, 128), h1)
    aggp1 = _agg_kernel(edge_index_curr, hs1.reshape(NPAD, D_HID), zeros2d)
    hs2 = _tc2(aggp1.reshape(NC, NPAD_P, 128), hs1, dis, b1t, w2b)
    aggp2 = _agg_kernel(edge_index_curr, hs2.reshape(NPAD, D_HID), zeros2d)
    out_p = _tc3(aggp2.reshape(NC, NPAD_P, 128), hs2, dis, b2t)
    return out_p.reshape(N, D_HID)
